# baseline (speedup 1.0000x reference)
import functools

import jax
import jax.numpy as jnp
from jax import lax
from jax.experimental import pallas as pl
from jax.experimental.pallas import tpu as pltpu
from jax.experimental.pallas import tpu_sc as plsc

_FEATURE_DIM = 64
_BOARD_SIZE = 15
_PCODE_DIM = 2380
_CELL_DIM = _BOARD_SIZE * _BOARD_SIZE
_CPAD = 240
_NCHUNK = _CPAD // 16


def _sc_embed(s, bd, offs, w_small, w_big, batch):
    info = plsc.get_sparse_core_info()
    nc, ns = info.num_cores, info.num_subcores
    nw = nc * ns
    bpw = batch // nw

    mesh = plsc.VectorSubcoreMesh(core_axis_name="c", subcore_axis_name="s")

    @functools.partial(
        pl.kernel,
        mesh=mesh,
        out_type=jax.ShapeDtypeStruct((batch, _FEATURE_DIM * _CELL_DIM),
                                      jnp.float32),
        compiler_params=pltpu.CompilerParams(
            use_tc_tiling_on_sc=False, needs_layout_passes=False),
        scratch_types=[
            pltpu.VMEM((2, _CPAD), jnp.int32),
            pltpu.VMEM((2, _CPAD), jnp.int32),
            pltpu.VMEM((_CPAD,), jnp.int32),
            pltpu.VMEM((2, 128), jnp.int32),
            pltpu.VMEM((2, 128), jnp.int32),
            pltpu.VMEM((2, 128), jnp.int32),
            pltpu.VMEM((2, 128), jnp.int32),
            pltpu.VMEM((256, _FEATURE_DIM), jnp.float32),
            pltpu.VMEM((256, _FEATURE_DIM), jnp.float32),
            pltpu.VMEM((256, _FEATURE_DIM), jnp.float32),
            pltpu.VMEM((256, _FEATURE_DIM), jnp.float32),
            pltpu.VMEM((_FEATURE_DIM * _CELL_DIM,), jnp.float32),
            pltpu.SemaphoreType.DMA,
        ],
    )
    def k(s_hbm, bd_hbm, offs_hbm, wsm_hbm, wbg_hbm, out_hbm,
          s_v, bd_v, offs_v, ix0, ix1, ix2, ix3, a0, a1, a2, a3, out_v, sem):
        wid = lax.axis_index("s") * nc + lax.axis_index("c")
        pltpu.sync_copy(offs_hbm, offs_v)
        zero16 = jnp.zeros((16,), jnp.int32)
        for ix in (ix0, ix1, ix2, ix3):
            ix[1, pl.ds(112, 16)] = zero16
        iota225 = lax.broadcasted_iota(jnp.int32, (16,), 0) * _CELL_DIM

        def batch_body(i, carry):
            b = wid * bpw + i
            pltpu.sync_copy(s_hbm.at[b], s_v)
            pltpu.sync_copy(bd_hbm.at[b], bd_v)
            for c in range(_NCHUNK):
                row, col = (0, c * 16) if c < 8 else (1, (c - 8) * 16)
                sl = pl.ds(c * 16, 16)
                ne = (bd_v[0, sl] + bd_v[1, sl]) > 0
                fill = jnp.full((16,), _PCODE_DIM, jnp.int32)
                i0 = jnp.where(ne, fill, s_v[0, sl])
                i1 = jnp.where(ne, fill, s_v[1, sl]) + (_PCODE_DIM + 1)
                off = offs_v[sl]
                dsl = pl.ds(col, 16)
                ix0[row, dsl] = i0
                ix1[row, dsl] = i1
                ix2[row, dsl] = i0 + off
                ix3[row, dsl] = i1 + off
            copies = []
            for ix, a, w in ((ix0, a0, wsm_hbm), (ix1, a1, wsm_hbm),
                             (ix2, a2, wbg_hbm), (ix3, a3, wbg_hbm)):
                for j in range(2):
                    copies.append(pltpu.async_copy(
                        w.at[ix.at[j]], a.at[pl.ds(j * 128, 128)], sem))
            for cp in copies:
                cp.wait()

            def cell_body(cc, carry2):
                for kk in range(_FEATURE_DIM // 16):
                    fs = pl.ds(kk * 16, 16)
                    v = a0[cc, fs] + a1[cc, fs] + a2[cc, fs] + a3[cc, fs]
                    plsc.store_scatter(
                        out_v, [iota225 + (kk * 16 * _CELL_DIM + cc)], v)
                return carry2
            lax.fori_loop(0, _CELL_DIM, cell_body, 0)
            pltpu.sync_copy(out_v, out_hbm.at[b])
            return carry

        lax.fori_loop(0, bpw, batch_body, 0)

    return k(s, bd, offs, w_small, w_big)


def kernel(sparse_feature_input, sparse_feature_dim, board_input,
           pcode_embedding_W, pcode_board_embedding_W, board_offset):
    del sparse_feature_dim
    batch = sparse_feature_input.shape[0]
    pad = _CPAD - _CELL_DIM
    s = sparse_feature_input[:, 10:12].reshape(batch, 2, _CELL_DIM)
    s = jnp.pad(s, ((0, 0), (0, 0), (0, pad)))
    bd = board_input.reshape(batch, 2, _CELL_DIM)
    bd = jnp.pad(bd, ((0, 0), (0, 0), (0, pad)))
    offs = jnp.pad(board_offset.reshape(_CELL_DIM), ((0, pad),))
    out = _sc_embed(s, bd, offs, pcode_embedding_W, pcode_board_embedding_W,
                    batch)
    return out.reshape(batch, _FEATURE_DIM, _BOARD_SIZE, _BOARD_SIZE)


# speedup vs baseline: 1.0294x; 1.0294x over previous
import functools

import jax
import jax.numpy as jnp
from jax import lax
from jax.experimental import pallas as pl
from jax.experimental.pallas import tpu as pltpu
from jax.experimental.pallas import tpu_sc as plsc

_FEATURE_DIM = 64
_BOARD_SIZE = 15
_PCODE_DIM = 2380
_CELL_DIM = _BOARD_SIZE * _BOARD_SIZE
_CPAD = 240
_NCHUNK = _CPAD // 16


def _sc_embed(s, bd, offs, w_small, w_big, batch):
    info = plsc.get_sparse_core_info()
    nc, ns = info.num_cores, info.num_subcores
    nw = nc * ns
    bpw = batch // nw

    mesh = plsc.VectorSubcoreMesh(core_axis_name="c", subcore_axis_name="s")

    @functools.partial(
        pl.kernel,
        mesh=mesh,
        out_type=jax.ShapeDtypeStruct((batch, _FEATURE_DIM * _CELL_DIM),
                                      jnp.float32),
        compiler_params=pltpu.CompilerParams(
            use_tc_tiling_on_sc=False, needs_layout_passes=False),
        scratch_types=[
            pltpu.VMEM((2, _CPAD), jnp.int32),
            pltpu.VMEM((2, _CPAD), jnp.int32),
            pltpu.VMEM((_CPAD,), jnp.int32),
            pltpu.VMEM((2, 128), jnp.int32),
            pltpu.VMEM((2, 128), jnp.int32),
            pltpu.VMEM((2, 128), jnp.int32),
            pltpu.VMEM((2, 128), jnp.int32),
            pltpu.VMEM((256, _FEATURE_DIM), jnp.float32),
            pltpu.VMEM((256, _FEATURE_DIM), jnp.float32),
            pltpu.VMEM((256, _FEATURE_DIM), jnp.float32),
            pltpu.VMEM((256, _FEATURE_DIM), jnp.float32),
            pltpu.VMEM((_FEATURE_DIM * _CELL_DIM,), jnp.float32),
            pltpu.SemaphoreType.DMA,
        ],
    )
    def k(s_hbm, bd_hbm, offs_hbm, wsm_hbm, wbg_hbm, out_hbm,
          s_v, bd_v, offs_v, ix0, ix1, ix2, ix3, a0, a1, a2, a3, out_v, sem):
        wid = lax.axis_index("s") * nc + lax.axis_index("c")
        pltpu.sync_copy(offs_hbm, offs_v)
        zero16 = jnp.zeros((16,), jnp.int32)
        for ix in (ix0, ix1, ix2, ix3):
            ix[1, pl.ds(112, 16)] = zero16
        iota225 = lax.broadcasted_iota(jnp.int32, (16,), 0) * _CELL_DIM

        def batch_body(i, carry):
            b = wid * bpw + i
            pltpu.sync_copy(s_hbm.at[b], s_v)
            pltpu.sync_copy(bd_hbm.at[b], bd_v)
            for c in range(_NCHUNK):
                row, col = (0, c * 16) if c < 8 else (1, (c - 8) * 16)
                sl = pl.ds(c * 16, 16)
                ne = (bd_v[0, sl] + bd_v[1, sl]) > 0
                fill = jnp.full((16,), _PCODE_DIM, jnp.int32)
                i0 = jnp.where(ne, fill, s_v[0, sl])
                i1 = jnp.where(ne, fill, s_v[1, sl]) + (_PCODE_DIM + 1)
                off = offs_v[sl]
                dsl = pl.ds(col, 16)
                ix0[row, dsl] = i0
                ix1[row, dsl] = i1
                ix2[row, dsl] = i0 + off
                ix3[row, dsl] = i1 + off
            copies = []
            for ix, a, w in ((ix0, a0, wsm_hbm), (ix1, a1, wsm_hbm),
                             (ix2, a2, wbg_hbm), (ix3, a3, wbg_hbm)):
                for j in range(2):
                    copies.append(pltpu.async_copy(
                        w.at[ix.at[j]], a.at[pl.ds(j * 128, 128)], sem))
            for cp in copies:
                cp.wait()

            def cell_body(cc, carry2):
                for kk in range(_FEATURE_DIM // 16):
                    fs = pl.ds(kk * 16, 16)
                    v = a0[cc, fs] + a1[cc, fs] + a2[cc, fs] + a3[cc, fs]
                    plsc.store_scatter(
                        out_v, [iota225 + (kk * 16 * _CELL_DIM + cc)], v)
                return carry2
            lax.fori_loop(0, _CELL_DIM, cell_body, 0)
            return carry

        lax.fori_loop(0, bpw, batch_body, 0)
        pltpu.sync_copy(out_v, out_hbm.at[wid])

    return k(s, bd, offs, w_small, w_big)


def kernel(sparse_feature_input, sparse_feature_dim, board_input,
           pcode_embedding_W, pcode_board_embedding_W, board_offset):
    del sparse_feature_dim
    batch = sparse_feature_input.shape[0]
    pad = _CPAD - _CELL_DIM
    s = sparse_feature_input[:, 10:12].reshape(batch, 2, _CELL_DIM)
    s = jnp.pad(s, ((0, 0), (0, 0), (0, pad)))
    bd = board_input.reshape(batch, 2, _CELL_DIM)
    bd = jnp.pad(bd, ((0, 0), (0, 0), (0, pad)))
    offs = jnp.pad(board_offset.reshape(_CELL_DIM), ((0, pad),))
    out = _sc_embed(s, bd, offs, pcode_embedding_W, pcode_board_embedding_W,
                    batch)
    return out.reshape(batch, _FEATURE_DIM, _BOARD_SIZE, _BOARD_SIZE)


# speedup vs baseline: 1.7805x; 1.7297x over previous
import functools

import jax
import jax.numpy as jnp
from jax import lax
from jax.experimental import pallas as pl
from jax.experimental.pallas import tpu as pltpu
from jax.experimental.pallas import tpu_sc as plsc

_FEATURE_DIM = 64
_BOARD_SIZE = 15
_PCODE_DIM = 2380
_CELL_DIM = _BOARD_SIZE * _BOARD_SIZE
_CPAD = 240
_NCHUNK = _CPAD // 16


def _sc_embed(s, bd, offs, w_small, w_big, batch):
    info = plsc.get_sparse_core_info()
    nc, ns = info.num_cores, info.num_subcores
    nw = nc * ns
    bpw = batch // nw

    mesh = plsc.VectorSubcoreMesh(core_axis_name="c", subcore_axis_name="s")

    @functools.partial(
        pl.kernel,
        mesh=mesh,
        out_type=jax.ShapeDtypeStruct((batch, _FEATURE_DIM * _CELL_DIM),
                                      jnp.float32),
        compiler_params=pltpu.CompilerParams(
            use_tc_tiling_on_sc=False, needs_layout_passes=False),
        scratch_types=[
            pltpu.VMEM((2, _CPAD), jnp.int32),
            pltpu.VMEM((2, _CPAD), jnp.int32),
            pltpu.VMEM((_CPAD,), jnp.int32),
            pltpu.VMEM((2, 128), jnp.int32),
            pltpu.VMEM((2, 128), jnp.int32),
            pltpu.VMEM((2, 128), jnp.int32),
            pltpu.VMEM((2, 128), jnp.int32),
            pltpu.VMEM((256, _FEATURE_DIM), jnp.float32),
            pltpu.VMEM((256, _FEATURE_DIM), jnp.float32),
            pltpu.VMEM((256, _FEATURE_DIM), jnp.float32),
            pltpu.VMEM((256, _FEATURE_DIM), jnp.float32),
            pltpu.VMEM((_FEATURE_DIM * _CELL_DIM,), jnp.float32),
            pltpu.SemaphoreType.DMA,
        ],
    )
    def k(s_hbm, bd_hbm, offs_hbm, wsm_hbm, wbg_hbm, out_hbm,
          s_v, bd_v, offs_v, ix0, ix1, ix2, ix3, a0, a1, a2, a3, out_v, sem):
        wid = lax.axis_index("s") * nc + lax.axis_index("c")
        pltpu.sync_copy(offs_hbm, offs_v)
        zero16 = jnp.zeros((16,), jnp.int32)
        for ix in (ix0, ix1, ix2, ix3):
            ix[1, pl.ds(112, 16)] = zero16
        iota225 = lax.broadcasted_iota(jnp.int32, (16,), 0) * _CELL_DIM

        def batch_body(i, carry):
            b = wid * bpw + i
            pltpu.sync_copy(s_hbm.at[b], s_v)
            pltpu.sync_copy(bd_hbm.at[b], bd_v)
            for c in range(_NCHUNK):
                row, col = (0, c * 16) if c < 8 else (1, (c - 8) * 16)
                sl = pl.ds(c * 16, 16)
                ne = (bd_v[0, sl] + bd_v[1, sl]) > 0
                fill = jnp.full((16,), _PCODE_DIM, jnp.int32)
                i0 = jnp.where(ne, fill, s_v[0, sl])
                i1 = jnp.where(ne, fill, s_v[1, sl]) + (_PCODE_DIM + 1)
                off = offs_v[sl]
                dsl = pl.ds(col, 16)
                ix0[row, dsl] = i0
                ix1[row, dsl] = i1
                ix2[row, dsl] = i0 + off
                ix3[row, dsl] = i1 + off
            copies = []
            for ix, a, w in ((ix0, a0, wsm_hbm), (ix1, a1, wsm_hbm),
                             (ix2, a2, wbg_hbm), (ix3, a3, wbg_hbm)):
                for j in range(1):
                    copies.append(pltpu.async_copy(
                        w.at[ix.at[j]], a.at[pl.ds(j * 128, 128)], sem))
            for cp in copies:
                cp.wait()

            def cell_body(cc, carry2):
                for kk in range(_FEATURE_DIM // 16):
                    fs = pl.ds(kk * 16, 16)
                    v = a0[cc, fs] + a1[cc, fs] + a2[cc, fs] + a3[cc, fs]
                    plsc.store_scatter(
                        out_v, [iota225 + (kk * 16 * _CELL_DIM + cc)], v)
                return carry2
            lax.fori_loop(0, _CELL_DIM, cell_body, 0)
            return carry

        lax.fori_loop(0, bpw, batch_body, 0)
        pltpu.sync_copy(out_v, out_hbm.at[wid])

    return k(s, bd, offs, w_small, w_big)


def kernel(sparse_feature_input, sparse_feature_dim, board_input,
           pcode_embedding_W, pcode_board_embedding_W, board_offset):
    del sparse_feature_dim
    batch = sparse_feature_input.shape[0]
    pad = _CPAD - _CELL_DIM
    s = sparse_feature_input[:, 10:12].reshape(batch, 2, _CELL_DIM)
    s = jnp.pad(s, ((0, 0), (0, 0), (0, pad)))
    bd = board_input.reshape(batch, 2, _CELL_DIM)
    bd = jnp.pad(bd, ((0, 0), (0, 0), (0, pad)))
    offs = jnp.pad(board_offset.reshape(_CELL_DIM), ((0, pad),))
    out = _sc_embed(s, bd, offs, pcode_embedding_W, pcode_board_embedding_W,
                    batch)
    return out.reshape(batch, _FEATURE_DIM, _BOARD_SIZE, _BOARD_SIZE)


# speedup vs baseline: 1.7934x; 1.0072x over previous
import functools

import jax
import jax.numpy as jnp
from jax import lax
from jax.experimental import pallas as pl
from jax.experimental.pallas import tpu as pltpu
from jax.experimental.pallas import tpu_sc as plsc

_FEATURE_DIM = 64
_BOARD_SIZE = 15
_PCODE_DIM = 2380
_CELL_DIM = _BOARD_SIZE * _BOARD_SIZE
_CPAD = 240
_NCHUNK = _CPAD // 16


def _sc_embed(s, bd, offs, w_small, w_big, batch):
    info = plsc.get_sparse_core_info()
    nc, ns = info.num_cores, info.num_subcores
    nw = nc * ns
    bpw = batch // nw

    mesh = plsc.VectorSubcoreMesh(core_axis_name="c", subcore_axis_name="s")

    @functools.partial(
        pl.kernel,
        mesh=mesh,
        out_type=jax.ShapeDtypeStruct((batch, _FEATURE_DIM * _CELL_DIM),
                                      jnp.float32),
        compiler_params=pltpu.CompilerParams(
            use_tc_tiling_on_sc=False, needs_layout_passes=False),
        scratch_types=[
            pltpu.VMEM((2, _CPAD), jnp.int32),
            pltpu.VMEM((2, _CPAD), jnp.int32),
            pltpu.VMEM((_CPAD,), jnp.int32),
            pltpu.VMEM((2, 128), jnp.int32),
            pltpu.VMEM((2, 128), jnp.int32),
            pltpu.VMEM((2, 128), jnp.int32),
            pltpu.VMEM((2, 128), jnp.int32),
            pltpu.VMEM((256, _FEATURE_DIM), jnp.float32),
            pltpu.VMEM((256, _FEATURE_DIM), jnp.float32),
            pltpu.VMEM((256, _FEATURE_DIM), jnp.float32),
            pltpu.VMEM((256, _FEATURE_DIM), jnp.float32),
            pltpu.VMEM((_FEATURE_DIM * _CELL_DIM,), jnp.float32),
            pltpu.SemaphoreType.DMA,
        ],
    )
    def k(s_hbm, bd_hbm, offs_hbm, wsm_hbm, wbg_hbm, out_hbm,
          s_v, bd_v, offs_v, ix0, ix1, ix2, ix3, a0, a1, a2, a3, out_v, sem):
        wid = lax.axis_index("s") * nc + lax.axis_index("c")
        pltpu.sync_copy(offs_hbm, offs_v)
        zero16 = jnp.zeros((16,), jnp.int32)
        for ix in (ix0, ix1, ix2, ix3):
            ix[1, pl.ds(112, 16)] = zero16
        iota225 = lax.broadcasted_iota(jnp.int32, (16,), 0) * _CELL_DIM

        def batch_body(i, carry):
            b = wid * bpw + i
            pltpu.sync_copy(s_hbm.at[b], s_v)
            pltpu.sync_copy(bd_hbm.at[b], bd_v)
            for c in range(_NCHUNK):
                row, col = (0, c * 16) if c < 8 else (1, (c - 8) * 16)
                sl = pl.ds(c * 16, 16)
                ne = (bd_v[0, sl] + bd_v[1, sl]) > 0
                fill = jnp.full((16,), _PCODE_DIM, jnp.int32)
                i0 = jnp.where(ne, fill, s_v[0, sl])
                i1 = jnp.where(ne, fill, s_v[1, sl]) + (_PCODE_DIM + 1)
                off = offs_v[sl]
                dsl = pl.ds(col, 16)
                ix0[row, dsl] = i0
                ix1[row, dsl] = i1
                ix2[row, dsl] = i0 + off
                ix3[row, dsl] = i1 + off
            copies = []
            for ix, a, w in ((ix0, a0, wsm_hbm), (ix1, a1, wsm_hbm),
                             (ix2, a2, wbg_hbm), (ix3, a3, wbg_hbm)):
                for j in range(2):
                    copies.append(pltpu.async_copy(
                        w.at[ix.at[j, pl.ds(0, 64)]],
                        a.at[pl.ds(j * 128, 64)], sem))
            for cp in copies:
                cp.wait()

            def cell_body(cc, carry2):
                for kk in range(_FEATURE_DIM // 16):
                    fs = pl.ds(kk * 16, 16)
                    v = a0[cc, fs] + a1[cc, fs] + a2[cc, fs] + a3[cc, fs]
                    plsc.store_scatter(
                        out_v, [iota225 + (kk * 16 * _CELL_DIM + cc)], v)
                return carry2
            lax.fori_loop(0, _CELL_DIM, cell_body, 0)
            return carry

        lax.fori_loop(0, bpw, batch_body, 0)
        pltpu.sync_copy(out_v, out_hbm.at[wid])

    return k(s, bd, offs, w_small, w_big)


def kernel(sparse_feature_input, sparse_feature_dim, board_input,
           pcode_embedding_W, pcode_board_embedding_W, board_offset):
    del sparse_feature_dim
    batch = sparse_feature_input.shape[0]
    pad = _CPAD - _CELL_DIM
    s = sparse_feature_input[:, 10:12].reshape(batch, 2, _CELL_DIM)
    s = jnp.pad(s, ((0, 0), (0, 0), (0, pad)))
    bd = board_input.reshape(batch, 2, _CELL_DIM)
    bd = jnp.pad(bd, ((0, 0), (0, 0), (0, pad)))
    offs = jnp.pad(board_offset.reshape(_CELL_DIM), ((0, pad),))
    out = _sc_embed(s, bd, offs, pcode_embedding_W, pcode_board_embedding_W,
                    batch)
    return out.reshape(batch, _FEATURE_DIM, _BOARD_SIZE, _BOARD_SIZE)


# speedup vs baseline: 4.9648x; 2.7684x over previous
import functools

import jax
import jax.numpy as jnp
from jax import lax
from jax.experimental import pallas as pl
from jax.experimental.pallas import tpu as pltpu
from jax.experimental.pallas import tpu_sc as plsc

_FEATURE_DIM = 64
_BOARD_SIZE = 15
_PCODE_DIM = 2380
_EMBED_DIM = 2 * (_PCODE_DIM + 1)
_CELL_DIM = _BOARD_SIZE * _BOARD_SIZE
_CPAD = 240
_LISTCAP = 256
_CHUNK = 32


def _sc_embed(s, bd, offs, w_small, w_big, batch):
    info = plsc.get_sparse_core_info()
    nc, ns = info.num_cores, info.num_subcores
    nw = nc * ns
    bpw = batch // nw
    n_f = _FEATURE_DIM * _CELL_DIM

    mesh = plsc.VectorSubcoreMesh(core_axis_name="c", subcore_axis_name="s")

    @functools.partial(
        pl.kernel,
        mesh=mesh,
        out_type=jax.ShapeDtypeStruct((batch, n_f), jnp.float32),
        compiler_params=pltpu.CompilerParams(
            use_tc_tiling_on_sc=False, needs_layout_passes=False),
        scratch_types=[
            pltpu.VMEM((2, _CPAD), jnp.int32),
            pltpu.VMEM((2, _CPAD), jnp.int32),
            pltpu.VMEM((_CPAD,), jnp.int32),
            pltpu.VMEM((_LISTCAP,), jnp.int32),
            pltpu.VMEM((_LISTCAP,), jnp.int32),
            pltpu.VMEM((_LISTCAP,), jnp.int32),
            pltpu.VMEM((_LISTCAP,), jnp.int32),
            pltpu.VMEM((_LISTCAP,), jnp.int32),
            pltpu.VMEM((_LISTCAP, _FEATURE_DIM), jnp.float32),
            pltpu.VMEM((_LISTCAP, _FEATURE_DIM), jnp.float32),
            pltpu.VMEM((_LISTCAP, _FEATURE_DIM), jnp.float32),
            pltpu.VMEM((_LISTCAP, _FEATURE_DIM), jnp.float32),
            pltpu.VMEM((n_f,), jnp.float32),
            pltpu.VMEM((n_f,), jnp.float32),
            pltpu.VMEM((_CPAD, _FEATURE_DIM), jnp.float32),
            pltpu.VMEM((16,), jnp.int32),
            pltpu.VMEM((16,), jnp.int32),
            pltpu.VMEM((2, _FEATURE_DIM), jnp.float32),
            pltpu.VMEM((2, _FEATURE_DIM), jnp.float32),
            pltpu.VMEM((_FEATURE_DIM,), jnp.float32),
            pltpu.VMEM_SHARED((_CPAD, _FEATURE_DIM), jnp.float32),
            pltpu.SMEM((1,), jnp.int32),
            pltpu.SemaphoreType.DMA,
        ],
    )
    def k(s_hbm, bd_hbm, offs_hbm, wsm_hbm, wbg_hbm, out_hbm,
          s_v, bd_v, offs_v, l0, l1, l2, l3, cid, g0, g1, g2, g3,
          out_v, const_t, const_v, idx2s, idx2b, rows2s, rows2b, crow,
          const_sp, n_ref, sem):
        cidx = lax.axis_index("c")
        sid = lax.axis_index("s")
        wid = sid * nc + cidx
        pltpu.sync_copy(offs_hbm, offs_v)
        zero16 = jnp.zeros((16,), jnp.int32)
        for lst in (l0, l1, l2, l3):
            for q in range(_LISTCAP // 16):
                lst[pl.ds(q * 16, 16)] = zero16
        iota16 = lax.broadcasted_iota(jnp.int32, (16,), 0)
        iota225 = iota16 * _CELL_DIM

        even = (iota16 % 2) == 0
        idx2s[...] = jnp.where(even, jnp.full((16,), _PCODE_DIM, jnp.int32),
                               jnp.full((16,), 2 * _PCODE_DIM + 1, jnp.int32))
        pltpu.async_copy(wsm_hbm.at[idx2s.at[pl.ds(0, 2)]], rows2s,
                         sem).wait()

        def const_body(i, carry):
            c = sid * 15 + i

            @pl.when(c < _CELL_DIM)
            def _():
                off = offs_v[pl.ds(c, 16)][0]
                idx2b[...] = jnp.where(
                    even, off + _PCODE_DIM, off + 2 * _PCODE_DIM + 1)
                pltpu.async_copy(wbg_hbm.at[idx2b.at[pl.ds(0, 2)]], rows2b,
                                 sem).wait()
                for kk in range(_FEATURE_DIM // 16):
                    fs = pl.ds(kk * 16, 16)
                    crow[fs] = (rows2s[0, fs] + rows2s[1, fs]
                                + rows2b[0, fs] + rows2b[1, fs])
                pltpu.sync_copy(crow, const_sp.at[c])
            return carry

        lax.fori_loop(0, 15, const_body, 0)
        plsc.subcore_barrier()
        pltpu.sync_copy(const_sp, const_v)

        def tr_body(c, carry):
            for kk in range(_FEATURE_DIM // 16):
                v = const_v[c, pl.ds(kk * 16, 16)]
                plsc.store_scatter(
                    const_t, [iota225 + (kk * 16 * _CELL_DIM + c)], v)
            return carry

        lax.fori_loop(0, _CELL_DIM, tr_body, 0)

        def batch_body(i, carry):
            b = wid * bpw + i
            pltpu.sync_copy(s_hbm.at[b], s_v)
            pltpu.sync_copy(bd_hbm.at[b], bd_v)

            def compact_body(q, nvec):
                sl = pl.ds(q * 16, 16)
                empty = (bd_v[0, sl] + bd_v[1, sl]) == 0
                cum = plsc.cumsum(jnp.where(empty, 1, 0).astype(jnp.int32))
                cnt = jnp.max(cum)
                pos = nvec + cum - 1
                p0 = s_v[0, sl]
                p1 = s_v[1, sl] + (_PCODE_DIM + 1)
                off = offs_v[sl]
                plsc.store_scatter(l0, [pos], p0, mask=empty)
                plsc.store_scatter(l1, [pos], p1, mask=empty)
                plsc.store_scatter(l2, [pos], p0 + off, mask=empty)
                plsc.store_scatter(l3, [pos], p1 + off, mask=empty)
                plsc.store_scatter(cid, [pos], iota16 + q * 16, mask=empty)
                return nvec + cnt

            nvec = lax.fori_loop(0, _CPAD // 16, compact_body,
                                 jnp.zeros((16,), jnp.int32))
            n = jnp.max(nvec)
            nch = (n + (_CHUNK - 1)) // _CHUNK

            def issue_body(kc, carry2):
                o = pl.multiple_of(kc * _CHUNK, _CHUNK)
                for lst, g, w in ((l0, g0, wsm_hbm), (l1, g1, wsm_hbm),
                                  (l2, g2, wbg_hbm), (l3, g3, wbg_hbm)):
                    pltpu.async_copy(
                        w.at[lst.at[pl.ds(o, _CHUNK)]],
                        g.at[pl.ds(o, _CHUNK)], sem)
                return carry2

            lax.fori_loop(0, nch, issue_body, 0)

            def copy_body(q, carry2):
                for u in range(4):
                    sl = pl.ds(q * 64 + u * 16, 16)
                    out_v[sl] = const_t[sl]
                return carry2

            lax.fori_loop(0, n_f // 64, copy_body, 0)

            def drain_body(kc, carry2):
                o = pl.multiple_of(kc * _CHUNK, _CHUNK)
                for lst, g, w in ((l0, g0, wsm_hbm), (l1, g1, wsm_hbm),
                                  (l2, g2, wbg_hbm), (l3, g3, wbg_hbm)):
                    pltpu.make_async_copy(
                        w.at[lst.at[pl.ds(o, _CHUNK)]],
                        g.at[pl.ds(o, _CHUNK)], sem).wait()
                return carry2

            lax.fori_loop(0, nch, drain_body, 0)

            def fix_body(j, carry2):
                c = cid[pl.ds(j, 16)][0]
                for kk in range(_FEATURE_DIM // 16):
                    fs = pl.ds(kk * 16, 16)
                    v = g0[j, fs] + g1[j, fs] + g2[j, fs] + g3[j, fs]
                    plsc.store_scatter(
                        out_v, [iota225 + (kk * 16 * _CELL_DIM + c)], v)
                return carry2

            lax.fori_loop(0, n, fix_body, 0)
            pltpu.sync_copy(out_v, out_hbm.at[b])
            return carry

        lax.fori_loop(0, bpw, batch_body, 0)

    return k(s, bd, offs, w_small, w_big)


def kernel(sparse_feature_input, sparse_feature_dim, board_input,
           pcode_embedding_W, pcode_board_embedding_W, board_offset):
    del sparse_feature_dim
    batch = sparse_feature_input.shape[0]
    pad = _CPAD - _CELL_DIM
    s = sparse_feature_input[:, 10:12].reshape(batch, 2, _CELL_DIM)
    s = jnp.pad(s, ((0, 0), (0, 0), (0, pad)))
    bd = board_input.reshape(batch, 2, _CELL_DIM)
    bd = jnp.pad(bd, ((0, 0), (0, 0), (0, pad)), constant_values=1)
    offs = jnp.pad(board_offset.reshape(_CELL_DIM), ((0, pad),))
    out = _sc_embed(s, bd, offs, pcode_embedding_W, pcode_board_embedding_W,
                    batch)
    return out.reshape(batch, _FEATURE_DIM, _BOARD_SIZE, _BOARD_SIZE)
